# Initial kernel scaffold; baseline (speedup 1.0000x reference)
import jax
import jax.numpy as jnp
from jax.experimental import pallas as pl


def kernel(feats, segment_ids, local_W, local_b, global_W, global_b):
    raise NotImplementedError("write your pallas kernel here")


# speedup vs baseline: 2.7931x; 2.7931x over previous
import jax
import jax.numpy as jnp
from jax.experimental import pallas as pl
from jax.experimental.pallas import tpu as pltpu

_N = 8192
_D = 512
_NSEG = 8
_BLK = 1024
_NCHUNK = _N // _BLK


def _seg_mask(seg_ref, i):
    segb = seg_ref[pl.ds(i * _BLK, _BLK), :]
    return segb == jax.lax.broadcasted_iota(jnp.int32, (_BLK, _NSEG), 1)


def _ain_body(feats_ref, seg_ref, w2_ref, b2_ref, out_ref, z_ref, u_ref):
    w2 = w2_ref[:]
    b2 = b2_ref[:]

    gmax = jnp.full((1, _NSEG), -1e30, jnp.float32)
    for i in range(_NCHUNK):
        fb = feats_ref[pl.ds(i * _BLK, _BLK), :]
        zb = jnp.dot(fb, w2, preferred_element_type=jnp.float32) + b2
        z_ref[pl.ds(i * _BLK, _BLK), :] = zb
        maskb = _seg_mask(seg_ref, i)
        gb = jnp.where(maskb, zb[:, 1:2], jnp.float32(-1e30))
        gmax = jnp.maximum(gmax, jnp.max(gb, axis=0, keepdims=True))

    denom = jnp.zeros((1, _NSEG), jnp.float32)
    for i in range(_NCHUNK):
        zb = z_ref[pl.ds(i * _BLK, _BLK), :]
        maskb = _seg_mask(seg_ref, i)
        onehot = maskb.astype(jnp.float32)
        gmax_row = jnp.sum(onehot * gmax, axis=1, keepdims=True)
        egb = jnp.exp(zb[:, 1:2] - gmax_row)
        denom = denom + jnp.sum(onehot * egb, axis=0, keepdims=True)
        u_ref[pl.ds(i * _BLK, _BLK), :] = jax.nn.sigmoid(zb[:, 0:1]) * egb

    inv_denom = 1.0 / denom
    usum = jnp.zeros((), jnp.float32)
    macc = jnp.zeros((1, _D), jnp.float32)
    for i in range(_NCHUNK):
        maskb = _seg_mask(seg_ref, i)
        onehot = maskb.astype(jnp.float32)
        inv_denom_row = jnp.sum(onehot * inv_denom, axis=1, keepdims=True)
        ub = u_ref[pl.ds(i * _BLK, _BLK), :] * inv_denom_row
        u_ref[pl.ds(i * _BLK, _BLK), :] = ub
        usum = usum + jnp.sum(ub)
        fb = feats_ref[pl.ds(i * _BLK, _BLK), :]
        macc = macc + jax.lax.dot_general(
            ub, fb, (((0,), (0,)), ((), ())),
            preferred_element_type=jnp.float32)
    mean = macc / usum

    vacc = jnp.zeros((1, _D), jnp.float32)
    for i in range(_NCHUNK):
        fb = feats_ref[pl.ds(i * _BLK, _BLK), :]
        ub = u_ref[pl.ds(i * _BLK, _BLK), :]
        r = fb - mean
        vacc = vacc + jax.lax.dot_general(
            ub, r * r, (((0,), (0,)), ((), ())),
            preferred_element_type=jnp.float32)
    inv_std = jax.lax.rsqrt(vacc / usum)

    for i in range(_NCHUNK):
        fb = feats_ref[pl.ds(i * _BLK, _BLK), :]
        out_ref[pl.ds(i * _BLK, _BLK), :] = (fb - mean) * inv_std


def kernel(feats, segment_ids, local_W, local_b, global_W, global_b):
    w2 = jnp.concatenate([local_W, global_W], axis=1)
    b2 = jnp.concatenate([local_b, global_b])[None, :]
    seg = segment_ids.reshape(_N, 1)
    return pl.pallas_call(
        _ain_body,
        out_shape=jax.ShapeDtypeStruct((_N, _D), jnp.float32),
        scratch_shapes=[
            pltpu.VMEM((_N, 2), jnp.float32),
            pltpu.VMEM((_N, 1), jnp.float32),
        ],
    )(feats, seg, w2, b2)


# speedup vs baseline: 2.9679x; 1.0626x over previous
import jax
import jax.numpy as jnp
from jax.experimental import pallas as pl
from jax.experimental.pallas import tpu as pltpu

_N = 8192
_D = 512
_NSEG = 8
_BLK = 1024
_NCHUNK = _N // _BLK


def _seg_mask(seg_ref, i):
    segb = seg_ref[pl.ds(i * _BLK, _BLK), :]
    return segb == jax.lax.broadcasted_iota(jnp.int32, (_BLK, _NSEG), 1)


def _in_copy(feats_hbm, fvm, isem, i):
    return pltpu.make_async_copy(
        feats_hbm.at[pl.ds(i * _BLK, _BLK), :],
        fvm.at[pl.ds(i * _BLK, _BLK), :],
        isem.at[i])


def _out_copy(obuf, out_hbm, osem, i):
    return pltpu.make_async_copy(
        obuf.at[i % 2],
        out_hbm.at[pl.ds(i * _BLK, _BLK), :],
        osem.at[i % 2])


def _ain_body(feats_hbm, seg_ref, w2_ref, b2_ref, out_hbm,
              fvm, z_ref, u_ref, obuf, isem, osem):
    w2 = w2_ref[:]
    b2 = b2_ref[:]

    for i in range(_NCHUNK):
        _in_copy(feats_hbm, fvm, isem, i).start()

    gmax = jnp.full((1, _NSEG), -1e30, jnp.float32)
    for i in range(_NCHUNK):
        _in_copy(feats_hbm, fvm, isem, i).wait()
        fb = fvm[pl.ds(i * _BLK, _BLK), :]
        zb = jnp.dot(fb, w2, preferred_element_type=jnp.float32) + b2
        z_ref[pl.ds(i * _BLK, _BLK), :] = zb
        maskb = _seg_mask(seg_ref, i)
        gb = jnp.where(maskb, zb[:, 1:2], jnp.float32(-1e30))
        gmax = jnp.maximum(gmax, jnp.max(gb, axis=0, keepdims=True))

    denom = jnp.zeros((1, _NSEG), jnp.float32)
    for i in range(_NCHUNK):
        zb = z_ref[pl.ds(i * _BLK, _BLK), :]
        maskb = _seg_mask(seg_ref, i)
        onehot = maskb.astype(jnp.float32)
        gmax_row = jnp.sum(onehot * gmax, axis=1, keepdims=True)
        egb = jnp.exp(zb[:, 1:2] - gmax_row)
        denom = denom + jnp.sum(onehot * egb, axis=0, keepdims=True)
        u_ref[pl.ds(i * _BLK, _BLK), :] = jax.nn.sigmoid(zb[:, 0:1]) * egb

    inv_denom = 1.0 / denom
    usum = jnp.zeros((), jnp.float32)
    macc = jnp.zeros((1, _D), jnp.float32)
    vacc = jnp.zeros((1, _D), jnp.float32)
    dn = (((0,), (0,)), ((), ()))
    for i in range(_NCHUNK):
        maskb = _seg_mask(seg_ref, i)
        onehot = maskb.astype(jnp.float32)
        inv_denom_row = jnp.sum(onehot * inv_denom, axis=1, keepdims=True)
        ub = u_ref[pl.ds(i * _BLK, _BLK), :] * inv_denom_row
        usum = usum + jnp.sum(ub)
        fb = fvm[pl.ds(i * _BLK, _BLK), :]
        macc = macc + jax.lax.dot_general(
            ub, fb, dn, preferred_element_type=jnp.float32)
        vacc = vacc + jax.lax.dot_general(
            ub, fb * fb, dn, preferred_element_type=jnp.float32)
    mean = macc / usum
    inv_std = jax.lax.rsqrt(vacc / usum - mean * mean)

    for i in range(_NCHUNK):
        s = i % 2
        if i >= 2:
            _out_copy(obuf, out_hbm, osem, i - 2).wait()
        fb = fvm[pl.ds(i * _BLK, _BLK), :]
        obuf[s, :, :] = (fb - mean) * inv_std
        _out_copy(obuf, out_hbm, osem, i).start()
    for i in range(_NCHUNK - 2, _NCHUNK):
        _out_copy(obuf, out_hbm, osem, i).wait()


def kernel(feats, segment_ids, local_W, local_b, global_W, global_b):
    w2 = jnp.concatenate([local_W, global_W], axis=1)
    b2 = jnp.concatenate([local_b, global_b])[None, :]
    seg = segment_ids.reshape(_N, 1)
    return pl.pallas_call(
        _ain_body,
        out_shape=jax.ShapeDtypeStruct((_N, _D), jnp.float32),
        in_specs=[
            pl.BlockSpec(memory_space=pl.ANY),
            pl.BlockSpec(memory_space=pltpu.VMEM),
            pl.BlockSpec(memory_space=pltpu.VMEM),
            pl.BlockSpec(memory_space=pltpu.VMEM),
        ],
        out_specs=pl.BlockSpec(memory_space=pl.ANY),
        scratch_shapes=[
            pltpu.VMEM((_N, _D), jnp.float32),
            pltpu.VMEM((_N, 2), jnp.float32),
            pltpu.VMEM((_N, 1), jnp.float32),
            pltpu.VMEM((2, _BLK, _D), jnp.float32),
            pltpu.SemaphoreType.DMA((_NCHUNK,)),
            pltpu.SemaphoreType.DMA((2,)),
        ],
    )(feats, seg, w2, b2)


# speedup vs baseline: 4.9612x; 1.6716x over previous
import jax
import jax.numpy as jnp
from jax.experimental import pallas as pl
from jax.experimental.pallas import tpu as pltpu

_N = 8192
_D = 512
_NSEG = 8
_BLK = 1024
_NCHUNK = _N // _BLK


def _seg_onehot(seg_ref, i):
    segc = seg_ref[:, pl.ds(i * _BLK, _BLK)]
    return segc == jax.lax.broadcasted_iota(jnp.int32, (_NSEG, _BLK), 0)


def _in_copy(feats_hbm, fvm, isem, i):
    return pltpu.make_async_copy(
        feats_hbm.at[pl.ds(i * _BLK, _BLK), :],
        fvm.at[pl.ds(i * _BLK, _BLK), :],
        isem.at[i])


def _out_copy(obuf, out_hbm, osem, i):
    return pltpu.make_async_copy(
        obuf.at[i % 2],
        out_hbm.at[pl.ds(i * _BLK, _BLK), :],
        osem.at[i % 2])


def _ain_body(feats_hbm, seg_ref, w2t_ref, b2_ref, out_hbm,
              fvm, z_ref, u_ref, obuf, isem, osem):
    w2t = w2t_ref[:]
    b2 = b2_ref[:]

    for i in range(_NCHUNK):
        _in_copy(feats_hbm, fvm, isem, i).start()

    gmax = jnp.full((_NSEG, 1), -1e30, jnp.float32)
    for i in range(_NCHUNK):
        _in_copy(feats_hbm, fvm, isem, i).wait()
        fb = fvm[pl.ds(i * _BLK, _BLK), :]
        zt = jax.lax.dot_general(
            w2t, fb, (((1,), (1,)), ((), ())),
            preferred_element_type=jnp.float32) + b2
        z_ref[:, pl.ds(i * _BLK, _BLK)] = zt
        oh = _seg_onehot(seg_ref, i)
        gb = jnp.where(oh, zt[1:2, :], jnp.float32(-1e30))
        gmax = jnp.maximum(gmax, jnp.max(gb, axis=1, keepdims=True))

    denom = jnp.zeros((_NSEG, 1), jnp.float32)
    for i in range(_NCHUNK):
        zt = z_ref[:, pl.ds(i * _BLK, _BLK)]
        ohf = _seg_onehot(seg_ref, i).astype(jnp.float32)
        gmax_row = jnp.sum(ohf * gmax, axis=0, keepdims=True)
        eg = jnp.exp(zt[1:2, :] - gmax_row)
        denom = denom + jnp.sum(ohf * eg, axis=1, keepdims=True)
        u_ref[:, pl.ds(i * _BLK, _BLK)] = jax.nn.sigmoid(zt[0:1, :]) * eg

    inv_denom = 1.0 / denom
    usum = jnp.zeros((), jnp.float32)
    macc = jnp.zeros((1, _D), jnp.float32)
    vacc = jnp.zeros((1, _D), jnp.float32)
    for i in range(_NCHUNK):
        ohf = _seg_onehot(seg_ref, i).astype(jnp.float32)
        inv_denom_row = jnp.sum(ohf * inv_denom, axis=0, keepdims=True)
        u_row = u_ref[:, pl.ds(i * _BLK, _BLK)] * inv_denom_row
        usum = usum + jnp.sum(u_row)
        fb16 = fvm[pl.ds(i * _BLK, _BLK), :].astype(jnp.bfloat16)
        u16 = u_row.astype(jnp.bfloat16)
        macc = macc + jnp.dot(u16, fb16,
                              preferred_element_type=jnp.float32)
        vacc = vacc + jnp.dot(u16, fb16 * fb16,
                              preferred_element_type=jnp.float32)
    mean = macc / usum
    inv_std = jax.lax.rsqrt(vacc / usum - mean * mean)
    mshift = mean * inv_std

    for i in range(_NCHUNK):
        s = i % 2
        if i >= 2:
            _out_copy(obuf, out_hbm, osem, i - 2).wait()
        fb = fvm[pl.ds(i * _BLK, _BLK), :]
        obuf[s, :, :] = fb * inv_std - mshift
        _out_copy(obuf, out_hbm, osem, i).start()
    for i in range(_NCHUNK - 2, _NCHUNK):
        _out_copy(obuf, out_hbm, osem, i).wait()


def kernel(feats, segment_ids, local_W, local_b, global_W, global_b):
    w2t = jnp.concatenate([local_W, global_W], axis=1).T
    b2 = jnp.concatenate([local_b, global_b])[:, None]
    seg = segment_ids.reshape(1, _N)
    return pl.pallas_call(
        _ain_body,
        out_shape=jax.ShapeDtypeStruct((_N, _D), jnp.float32),
        in_specs=[
            pl.BlockSpec(memory_space=pl.ANY),
            pl.BlockSpec(memory_space=pltpu.VMEM),
            pl.BlockSpec(memory_space=pltpu.VMEM),
            pl.BlockSpec(memory_space=pltpu.VMEM),
        ],
        out_specs=pl.BlockSpec(memory_space=pl.ANY),
        scratch_shapes=[
            pltpu.VMEM((_N, _D), jnp.float32),
            pltpu.VMEM((2, _N), jnp.float32),
            pltpu.VMEM((1, _N), jnp.float32),
            pltpu.VMEM((2, _BLK, _D), jnp.float32),
            pltpu.SemaphoreType.DMA((_NCHUNK,)),
            pltpu.SemaphoreType.DMA((2,)),
        ],
    )(feats, seg, w2t, b2)


# speedup vs baseline: 5.0736x; 1.0227x over previous
import jax
import jax.numpy as jnp
from jax.experimental import pallas as pl
from jax.experimental.pallas import tpu as pltpu

_N = 8192
_D = 512
_NSEG = 8
_BLK = 1024
_NCHUNK = _N // _BLK


def _seg_onehot(seg_ref, i):
    segc = seg_ref[:, pl.ds(i * _BLK, _BLK)]
    return segc == jax.lax.broadcasted_iota(jnp.int32, (_NSEG, _BLK), 0)


def _in_copy(feats_hbm, fvm, isem, i):
    return pltpu.make_async_copy(
        feats_hbm.at[pl.ds(i * _BLK, _BLK), :],
        fvm.at[pl.ds(i * _BLK, _BLK), :],
        isem.at[i])


def _out_copy(obuf, out_hbm, osem, i):
    return pltpu.make_async_copy(
        obuf.at[i % 2],
        out_hbm.at[pl.ds(i * _BLK, _BLK), :],
        osem.at[i % 2])


def _ain_body(feats_hbm, seg_ref, w2t_ref, b2_ref, out_hbm,
              fvm, obuf, isem, osem):
    w2t16 = w2t_ref[:].astype(jnp.bfloat16)
    b2 = b2_ref[:]

    for i in range(_NCHUNK):
        _in_copy(feats_hbm, fvm, isem, i).start()

    dn_t = (((1,), (1,)), ((), ()))
    mseg = jnp.full((_NSEG, 1), -1e30, jnp.float32)
    dseg = jnp.zeros((_NSEG, 1), jnp.float32)
    qseg = jnp.zeros((_NSEG, 1), jnp.float32)
    pacc = jnp.zeros((_NSEG, _D), jnp.float32)
    vacc = jnp.zeros((_NSEG, _D), jnp.float32)
    for i in range(_NCHUNK):
        _in_copy(feats_hbm, fvm, isem, i).wait()
        fb16 = fvm[pl.ds(i * _BLK, _BLK), :].astype(jnp.bfloat16)
        zt = jax.lax.dot_general(
            w2t16, fb16, dn_t,
            preferred_element_type=jnp.float32) + b2
        oh = _seg_onehot(seg_ref, i)
        ohf = oh.astype(jnp.float32)
        gb = jnp.where(oh, zt[1:2, :], jnp.float32(-1e30))
        mnew = jnp.maximum(mseg, jnp.max(gb, axis=1, keepdims=True))
        alpha = jnp.exp(mseg - mnew)
        mseg = mnew
        goff = jnp.sum(ohf * mnew, axis=0, keepdims=True)
        eg = jnp.exp(zt[1:2, :] - goff)
        uh = jax.nn.sigmoid(zt[0:1, :]) * eg
        egm = ohf * eg
        uhm = ohf * uh
        dseg = dseg * alpha + jnp.sum(egm, axis=1, keepdims=True)
        qseg = qseg * alpha + jnp.sum(uhm, axis=1, keepdims=True)
        u16 = uhm.astype(jnp.bfloat16)
        pacc = pacc * alpha + jax.lax.dot_general(
            u16, fb16, (((1,), (0,)), ((), ())),
            preferred_element_type=jnp.float32)
        vacc = vacc * alpha + jax.lax.dot_general(
            u16, fb16 * fb16, (((1,), (0,)), ((), ())),
            preferred_element_type=jnp.float32)

    inv_d = jnp.where(dseg > 0, 1.0 / dseg, 0.0)
    usum = jnp.sum(qseg * inv_d)
    mean = jnp.sum(pacc * inv_d, axis=0, keepdims=True) / usum
    ex2 = jnp.sum(vacc * inv_d, axis=0, keepdims=True) / usum
    inv_std = jax.lax.rsqrt(ex2 - mean * mean)
    mshift = mean * inv_std

    for i in range(_NCHUNK):
        s = i % 2
        if i >= 2:
            _out_copy(obuf, out_hbm, osem, i - 2).wait()
        fb = fvm[pl.ds(i * _BLK, _BLK), :]
        obuf[s, :, :] = fb * inv_std - mshift
        _out_copy(obuf, out_hbm, osem, i).start()
    for i in range(_NCHUNK - 2, _NCHUNK):
        _out_copy(obuf, out_hbm, osem, i).wait()


def kernel(feats, segment_ids, local_W, local_b, global_W, global_b):
    w2t = jnp.concatenate([local_W, global_W], axis=1).T
    b2 = jnp.concatenate([local_b, global_b])[:, None]
    seg = segment_ids.reshape(1, _N)
    return pl.pallas_call(
        _ain_body,
        out_shape=jax.ShapeDtypeStruct((_N, _D), jnp.float32),
        in_specs=[
            pl.BlockSpec(memory_space=pl.ANY),
            pl.BlockSpec(memory_space=pltpu.VMEM),
            pl.BlockSpec(memory_space=pltpu.VMEM),
            pl.BlockSpec(memory_space=pltpu.VMEM),
        ],
        out_specs=pl.BlockSpec(memory_space=pl.ANY),
        scratch_shapes=[
            pltpu.VMEM((_N, _D), jnp.float32),
            pltpu.VMEM((2, _BLK, _D), jnp.float32),
            pltpu.SemaphoreType.DMA((_NCHUNK,)),
            pltpu.SemaphoreType.DMA((2,)),
        ],
    )(feats, seg, w2t, b2)
